# Initial kernel scaffold; baseline (speedup 1.0000x reference)
import jax
import jax.numpy as jnp
from jax.experimental import pallas as pl


def kernel(x, dictionary):
    raise NotImplementedError("write your pallas kernel here")


# speedup vs baseline: 2.3497x; 2.3497x over previous
import jax
import jax.numpy as jnp
from jax.experimental import pallas as pl

_NUM_EMBEDDINGS = 1024
_DIM = 64
_BLK = 512


def _vq_body(x_ref, d_ref, dt_ref, out_ref):
    xb = x_ref[...]
    dm = d_ref[...]
    sim = jnp.dot(xb, dm, preferred_element_type=jnp.float32)
    rn = jnp.sum(xb * xb, axis=1, keepdims=True)
    cn = jnp.sum(dm * dm, axis=0, keepdims=True)
    dist = rn + cn - 2.0 * sim
    minv = jnp.min(dist, axis=1, keepdims=True)
    iota = jax.lax.broadcasted_iota(jnp.int32, dist.shape, 1)
    masked = jnp.where(dist == minv, iota, _NUM_EMBEDDINGS)
    idx = jnp.min(masked, axis=1, keepdims=True)
    one_hot = (iota == idx).astype(jnp.float32)
    out_ref[...] = jnp.dot(one_hot, dt_ref[...], preferred_element_type=jnp.float32)


def kernel(x, dictionary):
    orig_shape = x.shape
    flat = x.reshape(-1, _DIM)
    b = flat.shape[0]
    dict_t = dictionary.T
    q = pl.pallas_call(
        _vq_body,
        grid=(b // _BLK,),
        in_specs=[
            pl.BlockSpec((_BLK, _DIM), lambda i: (i, 0)),
            pl.BlockSpec((_DIM, _NUM_EMBEDDINGS), lambda i: (0, 0)),
            pl.BlockSpec((_NUM_EMBEDDINGS, _DIM), lambda i: (0, 0)),
        ],
        out_specs=pl.BlockSpec((_BLK, _DIM), lambda i: (i, 0)),
        out_shape=jax.ShapeDtypeStruct((b, _DIM), jnp.float32),
    )(flat, dictionary, dict_t)
    return q.reshape(orig_shape)
